# baseline (speedup 1.0000x reference)
import functools

import jax
import jax.numpy as jnp
from jax import lax
from jax.experimental import pallas as pl
from jax.experimental.pallas import tpu as pltpu
from jax.experimental.pallas import tpu_sc as plsc

VOCAB = 1000000
D = 64
DP = 128
BATCH = 4096
SEQ = 200
N = BATCH * SEQ
SCALE = 8.0

NC = 2
NS = 16
NW = NC * NS

IW = 128
NJOBS = N // IW
JOBS_W = NJOBS // NW
BH = BATCH // IW

_mesh = plsc.VectorSubcoreMesh(core_axis_name="c", subcore_axis_name="s")

TV = 4096


def _tt_body(x_ref, o_ref):
    t = lax.transpose(x_ref[...], (1, 0)) * SCALE
    o_ref[...] = jnp.concatenate(
        [t, jnp.zeros((TV, DP - D), jnp.float32)], axis=1)


_tc_transpose = pl.pallas_call(
    _tt_body,
    grid=(pl.cdiv(VOCAB, TV),),
    in_specs=[pl.BlockSpec((D, TV), lambda i: (0, i))],
    out_specs=pl.BlockSpec((TV, DP), lambda i: (i, 0)),
    out_shape=jax.ShapeDtypeStruct((VOCAB, DP), jnp.float32),
)

_IOTA16 = None


@functools.partial(
    pl.kernel,
    mesh=_mesh,
    out_type=jax.ShapeDtypeStruct((SEQ, 8, BH, 8, IW), jnp.float32),
    compiler_params=pltpu.CompilerParams(
        use_tc_tiling_on_sc=False, needs_layout_passes=False),
    scratch_types=[
        pltpu.VMEM((JOBS_W, IW), jnp.int32),
        pltpu.VMEM((IW, DP), jnp.float32),
        pltpu.VMEM((IW, DP), jnp.float32),
        pltpu.VMEM((D, IW), jnp.float32),
        pltpu.VMEM((D, IW), jnp.float32),
        pltpu.SemaphoreType.DMA,
        pltpu.SemaphoreType.DMA,
        pltpu.SemaphoreType.DMA,
        pltpu.SemaphoreType.DMA,
        pltpu.SemaphoreType.DMA,
    ],
)
def _sc_embed(idx_hbm, table_hbm, out_hbm, idx_v, g_a, g_b, t_a, t_b,
              isem, gsem_a, gsem_b, ssem_a, ssem_b):
    wid = lax.axis_index("s") * NC + lax.axis_index("c")
    job0 = wid * JOBS_W
    pltpu.async_copy(idx_hbm.at[pl.ds(job0, JOBS_W)], idx_v, isem).wait()

    iota = lax.iota(jnp.int32, 16)

    def fire(t, buf, sem):
        pltpu.async_copy(table_hbm.at[idx_v.at[t]], buf, sem)

    def drain(t, buf, sem):
        pltpu.make_async_copy(table_hbm.at[idx_v.at[t]], buf, sem).wait()

    def transpose_scale(gbuf, tbuf):
        @pl.loop(0, 8)
        def _(jb):
            rows = iota + jb * 16
            col0 = jb * 16
            for d in range(D):
                v = plsc.load_gather(gbuf, [rows, jnp.full((16,), d, jnp.int32)])
                tbuf.at[d][pl.ds(col0, 16)] = v

    def fire_stores(t, tbuf, sem):
        j = job0 + t
        s = j // BH
        bh = j % BH
        for dh in range(8):
            pltpu.async_copy(
                tbuf.at[pl.ds(dh * 8, 8)], out_hbm.at[s].at[dh].at[bh], sem)

    def wait_stores(tbuf, sem):
        for dh in range(8):
            pltpu.make_async_copy(
                tbuf.at[pl.ds(dh * 8, 8)], out_hbm.at[0].at[dh].at[0], sem
            ).wait()

    fire(0, g_a, gsem_a)

    @pl.loop(0, JOBS_W // 2)
    def _(p):
        t0 = 2 * p
        fire(t0 + 1, g_b, gsem_b)
        drain(t0, g_a, gsem_a)

        @pl.when(p > 0)
        def _():
            wait_stores(t_a, ssem_a)

        transpose_scale(g_a, t_a)
        fire_stores(t0, t_a, ssem_a)

        @pl.when(t0 + 2 < JOBS_W)
        def _():
            fire(t0 + 2, g_a, gsem_a)

        drain(t0 + 1, g_b, gsem_b)

        @pl.when(p > 0)
        def _():
            wait_stores(t_b, ssem_b)

        transpose_scale(g_b, t_b)
        fire_stores(t0 + 1, t_b, ssem_b)

    wait_stores(t_a, ssem_a)
    wait_stores(t_b, ssem_b)


def kernel(x, table):
    table_t = jnp.swapaxes(table, 0, 1)
    table_p = _tc_transpose(table_t)
    idx = jnp.swapaxes(x, 0, 1).reshape(NJOBS, IW)
    out5 = _sc_embed(idx, table_p)
    out = out5.transpose(2, 4, 0, 1, 3).reshape(BATCH, SEQ, D)
    return out


# speedup vs baseline: 1.1289x; 1.1289x over previous
import functools

import jax
import jax.numpy as jnp
from jax import lax
from jax.experimental import pallas as pl
from jax.experimental.pallas import tpu as pltpu
from jax.experimental.pallas import tpu_sc as plsc

VOCAB = 1000000
D = 64
DP = 128
BATCH = 4096
SEQ = 200
N = BATCH * SEQ
SCALE = 8.0

NC = 2
NS = 16
NW = NC * NS

IW = 128
NJOBS = N // IW
JOBS_W = NJOBS // NW
BH = BATCH // IW

_mesh = plsc.VectorSubcoreMesh(core_axis_name="c", subcore_axis_name="s")

TV = 4096


def _tt_body(x_ref, o_ref):
    t = lax.transpose(x_ref[...], (1, 0)) * SCALE
    o_ref[...] = jnp.concatenate([t, t], axis=1)


_tc_transpose = pl.pallas_call(
    _tt_body,
    grid=(pl.cdiv(VOCAB, TV),),
    in_specs=[pl.BlockSpec((D, TV), lambda i: (0, i))],
    out_specs=pl.BlockSpec((TV, DP), lambda i: (i, 0)),
    out_shape=jax.ShapeDtypeStruct((VOCAB, DP), jnp.float32),
)

_IOTA16 = None


@functools.partial(
    pl.kernel,
    mesh=_mesh,
    out_type=jax.ShapeDtypeStruct((SEQ, 8, BH, 8, IW), jnp.float32),
    compiler_params=pltpu.CompilerParams(
        use_tc_tiling_on_sc=False, needs_layout_passes=False),
    scratch_types=[
        pltpu.VMEM((JOBS_W, IW), jnp.int32),
        pltpu.VMEM((IW, D), jnp.float32),
        pltpu.VMEM((IW, D), jnp.float32),
        pltpu.VMEM((D, IW), jnp.float32),
        pltpu.VMEM((D, IW), jnp.float32),
        pltpu.SemaphoreType.DMA,
        pltpu.SemaphoreType.DMA,
        pltpu.SemaphoreType.DMA,
        pltpu.SemaphoreType.DMA,
        pltpu.SemaphoreType.DMA,
    ],
)
def _sc_embed(idx_hbm, table_hbm, out_hbm, idx_v, g_a, g_b, t_a, t_b,
              isem, gsem_a, gsem_b, ssem_a, ssem_b):
    wid = lax.axis_index("s") * NC + lax.axis_index("c")
    job0 = wid * JOBS_W
    pltpu.async_copy(idx_hbm.at[pl.ds(job0, JOBS_W)], idx_v, isem).wait()

    iota = lax.iota(jnp.int32, 16)

    def fire(t, buf, sem):
        pltpu.async_copy(table_hbm.at[idx_v.at[t]], buf, sem)

    def drain(t, buf, sem):
        pltpu.make_async_copy(table_hbm.at[idx_v.at[t]], buf, sem).wait()

    rows_c = [iota + c * 16 for c in range(D // 16)]

    def transpose_scale(gbuf, tbuf):
        @pl.loop(0, IW, step=4)
        def _(r):
            for rr in range(4):
                col = jnp.full((16,), r + rr, jnp.int32)
                for c in range(D // 16):
                    v = gbuf.at[r + rr][pl.ds(c * 16, 16)]
                    plsc.store_scatter(tbuf, [rows_c[c], col], v)

    def fire_stores(t, tbuf, sem):
        j = job0 + t
        s = j // BH
        bh = j % BH
        for dh in range(8):
            pltpu.async_copy(
                tbuf.at[pl.ds(dh * 8, 8)], out_hbm.at[s].at[dh].at[bh], sem)

    def wait_stores(tbuf, sem):
        for dh in range(8):
            pltpu.make_async_copy(
                tbuf.at[pl.ds(dh * 8, 8)], out_hbm.at[0].at[dh].at[0], sem
            ).wait()

    fire(0, g_a, gsem_a)

    @pl.loop(0, JOBS_W // 2)
    def _(p):
        t0 = 2 * p
        fire(t0 + 1, g_b, gsem_b)
        drain(t0, g_a, gsem_a)

        @pl.when(p > 0)
        def _():
            wait_stores(t_a, ssem_a)

        transpose_scale(g_a, t_a)
        fire_stores(t0, t_a, ssem_a)

        @pl.when(t0 + 2 < JOBS_W)
        def _():
            fire(t0 + 2, g_a, gsem_a)

        drain(t0 + 1, g_b, gsem_b)

        @pl.when(p > 0)
        def _():
            wait_stores(t_b, ssem_b)

        transpose_scale(g_b, t_b)
        fire_stores(t0 + 1, t_b, ssem_b)

    wait_stores(t_a, ssem_a)
    wait_stores(t_b, ssem_b)


def kernel(x, table):
    table_t = jnp.swapaxes(table, 0, 1)
    table_p = _tc_transpose(table_t)
    table_rows = table_p.reshape(2 * VOCAB, D)
    idx = (jnp.swapaxes(x, 0, 1) * 2).reshape(NJOBS, IW)
    out5 = _sc_embed(idx, table_rows)
    out = out5.transpose(2, 4, 0, 1, 3).reshape(BATCH, SEQ, D)
    return out


# speedup vs baseline: 1.4203x; 1.2580x over previous
import functools

import jax
import jax.numpy as jnp
from jax import lax
from jax.experimental import pallas as pl
from jax.experimental.pallas import tpu as pltpu
from jax.experimental.pallas import tpu_sc as plsc

VOCAB = 1000000
D = 64
DP = 128
BATCH = 4096
SEQ = 200
N = BATCH * SEQ
SCALE = 8.0

NC = 2
NS = 16
NW = NC * NS

IW = 128
NJOBS = N // IW
JOBS_W = NJOBS // NW
BH = BATCH // IW

_mesh = plsc.VectorSubcoreMesh(core_axis_name="c", subcore_axis_name="s")

TV = 4096


def _tt_body(x_ref, o_ref):
    t = lax.transpose(x_ref[...], (1, 0)) * SCALE
    o_ref[...] = jnp.concatenate([t, t], axis=1)


_tc_transpose = pl.pallas_call(
    _tt_body,
    grid=(pl.cdiv(VOCAB, TV),),
    in_specs=[pl.BlockSpec((D, TV), lambda i: (0, i))],
    out_specs=pl.BlockSpec((TV, DP), lambda i: (i, 0)),
    out_shape=jax.ShapeDtypeStruct((VOCAB, DP), jnp.float32),
)

_IOTA16 = None


@functools.partial(
    pl.kernel,
    mesh=_mesh,
    out_type=jax.ShapeDtypeStruct((SEQ, 8, BH, 8, IW), jnp.float32),
    compiler_params=pltpu.CompilerParams(
        use_tc_tiling_on_sc=False, needs_layout_passes=False),
    scratch_types=[
        pltpu.VMEM((JOBS_W, IW), jnp.int32),
        pltpu.VMEM((IW, D), jnp.float32),
        pltpu.VMEM((IW, D), jnp.float32),
        pltpu.VMEM((D, IW), jnp.float32),
        pltpu.VMEM((D, IW), jnp.float32),
        pltpu.SemaphoreType.DMA,
        pltpu.SemaphoreType.DMA,
        pltpu.SemaphoreType.DMA,
        pltpu.SemaphoreType.DMA,
        pltpu.SemaphoreType.DMA,
    ],
)
def _sc_embed(idx_hbm, table_hbm, out_hbm, idx_v, g_a, g_b, t_a, t_b,
              isem, gsem_a, gsem_b, ssem_a, ssem_b):
    wid = lax.axis_index("s") * NC + lax.axis_index("c")
    job0 = wid * JOBS_W
    pltpu.async_copy(idx_hbm.at[pl.ds(job0, JOBS_W)], idx_v, isem).wait()

    iota = lax.iota(jnp.int32, 16)

    def fire(t, buf, sem):
        pltpu.async_copy(table_hbm.at[idx_v.at[t]], buf, sem)

    def drain(t, buf, sem):
        pltpu.make_async_copy(table_hbm.at[idx_v.at[t]], buf, sem).wait()

    rows_c = [iota + c * 16 for c in range(D // 16)]

    def transpose_scale(gbuf, tbuf):
        @plsc.parallel_loop(0, IW, unroll=8)
        def _(r):
            col = jnp.full((16,), r, jnp.int32)
            for c in range(D // 16):
                v = gbuf.at[r][pl.ds(c * 16, 16)]
                plsc.store_scatter(tbuf, [rows_c[c], col], v)

    def fire_stores(t, tbuf, sem):
        j = job0 + t
        s = j // BH
        bh = j % BH
        for dh in range(8):
            pltpu.async_copy(
                tbuf.at[pl.ds(dh * 8, 8)], out_hbm.at[s].at[dh].at[bh], sem)

    def wait_stores(tbuf, sem):
        for dh in range(8):
            pltpu.make_async_copy(
                tbuf.at[pl.ds(dh * 8, 8)], out_hbm.at[0].at[dh].at[0], sem
            ).wait()

    fire(0, g_a, gsem_a)

    @pl.loop(0, JOBS_W // 2)
    def _(p):
        t0 = 2 * p
        fire(t0 + 1, g_b, gsem_b)
        drain(t0, g_a, gsem_a)

        @pl.when(p > 0)
        def _():
            wait_stores(t_a, ssem_a)

        transpose_scale(g_a, t_a)
        fire_stores(t0, t_a, ssem_a)

        @pl.when(t0 + 2 < JOBS_W)
        def _():
            fire(t0 + 2, g_a, gsem_a)

        drain(t0 + 1, g_b, gsem_b)

        @pl.when(p > 0)
        def _():
            wait_stores(t_b, ssem_b)

        transpose_scale(g_b, t_b)
        fire_stores(t0 + 1, t_b, ssem_b)

    wait_stores(t_a, ssem_a)
    wait_stores(t_b, ssem_b)


def kernel(x, table):
    table_t = jnp.swapaxes(table, 0, 1)
    table_p = _tc_transpose(table_t)
    table_rows = table_p.reshape(2 * VOCAB, D)
    idx = (jnp.swapaxes(x, 0, 1) * 2).reshape(NJOBS, IW)
    out5 = _sc_embed(idx, table_rows)
    out = out5.transpose(2, 4, 0, 1, 3).reshape(BATCH, SEQ, D)
    return out


# speedup vs baseline: 3.0410x; 2.1412x over previous
import functools

import jax
import jax.numpy as jnp
from jax import lax
from jax.experimental import pallas as pl
from jax.experimental.pallas import tpu as pltpu
from jax.experimental.pallas import tpu_sc as plsc

VOCAB = 1000000
D = 64
DP = 128
BATCH = 4096
SEQ = 200
N = BATCH * SEQ
SCALE = 8.0

NC = 2
NS = 16
NW = NC * NS

IW = 128
NJOBS = N // IW
JOBS_W = NJOBS // NW
BH = BATCH // IW

_mesh = plsc.VectorSubcoreMesh(core_axis_name="c", subcore_axis_name="s")

TV = 8192


def _tt_body(x_ref, o_ref):
    t = lax.transpose(x_ref[...], (1, 0)) * SCALE
    o_ref[...] = jnp.concatenate([t, t], axis=1)


_tc_transpose = pl.pallas_call(
    _tt_body,
    grid=(pl.cdiv(VOCAB, TV),),
    in_specs=[pl.BlockSpec((D, TV), lambda i: (0, i))],
    out_specs=pl.BlockSpec((TV, DP), lambda i: (i, 0)),
    out_shape=jax.ShapeDtypeStruct((VOCAB, DP), jnp.float32),
)

_IOTA16 = None


@functools.partial(
    pl.kernel,
    mesh=_mesh,
    out_type=jax.ShapeDtypeStruct((SEQ, 8, BH, 8, IW), jnp.float32),
    compiler_params=pltpu.CompilerParams(
        use_tc_tiling_on_sc=False, needs_layout_passes=False),
    scratch_types=[
        pltpu.VMEM((JOBS_W, IW), jnp.int32),
        pltpu.VMEM((IW, D), jnp.float32),
        pltpu.VMEM((IW, D), jnp.float32),
        pltpu.VMEM((D, IW + 1), jnp.float32),
        pltpu.VMEM((D, IW + 1), jnp.float32),
        pltpu.SemaphoreType.DMA,
        pltpu.SemaphoreType.DMA,
        pltpu.SemaphoreType.DMA,
        pltpu.SemaphoreType.DMA,
        pltpu.SemaphoreType.DMA,
    ],
)
def _sc_embed(idx_hbm, table_hbm, out_hbm, idx_v, g_a, g_b, t_a, t_b,
              isem, gsem_a, gsem_b, ssem_a, ssem_b):
    wid = lax.axis_index("s") * NC + lax.axis_index("c")
    job0 = wid * JOBS_W
    pltpu.async_copy(idx_hbm.at[pl.ds(job0, JOBS_W)], idx_v, isem).wait()

    iota = lax.iota(jnp.int32, 16)

    def fire(t, buf, sem):
        pltpu.async_copy(table_hbm.at[idx_v.at[t]], buf, sem)

    def drain(t, buf, sem):
        pltpu.make_async_copy(table_hbm.at[idx_v.at[t]], buf, sem).wait()

    rows_c = [iota + c * 16 for c in range(D // 16)]

    def transpose_scale(gbuf, tbuf):
        @plsc.parallel_loop(0, IW, unroll=8)
        def _(r):
            col = jnp.full((16,), r, jnp.int32)
            for c in range(D // 16):
                v = gbuf.at[r][pl.ds(c * 16, 16)]
                plsc.store_scatter(tbuf, [rows_c[c], col], v)

    def fire_stores(t, tbuf, sem):
        j = job0 + t
        s = j // BH
        bh = j % BH
        for dh in range(8):
            pltpu.async_copy(
                tbuf.at[pl.ds(dh * 8, 8), pl.ds(0, IW)],
                out_hbm.at[s].at[dh].at[bh], sem)

    def wait_stores(tbuf, sem):
        for dh in range(8):
            pltpu.make_async_copy(
                tbuf.at[pl.ds(dh * 8, 8), pl.ds(0, IW)],
                out_hbm.at[0].at[dh].at[0], sem
            ).wait()

    fire(0, g_a, gsem_a)

    @pl.loop(0, JOBS_W // 2)
    def _(p):
        t0 = 2 * p
        fire(t0 + 1, g_b, gsem_b)
        drain(t0, g_a, gsem_a)

        @pl.when(p > 0)
        def _():
            wait_stores(t_a, ssem_a)

        transpose_scale(g_a, t_a)
        fire_stores(t0, t_a, ssem_a)

        @pl.when(t0 + 2 < JOBS_W)
        def _():
            fire(t0 + 2, g_a, gsem_a)

        drain(t0 + 1, g_b, gsem_b)

        @pl.when(p > 0)
        def _():
            wait_stores(t_b, ssem_b)

        transpose_scale(g_b, t_b)
        fire_stores(t0 + 1, t_b, ssem_b)

    wait_stores(t_a, ssem_a)
    wait_stores(t_b, ssem_b)


def kernel(x, table):
    table_t = jnp.swapaxes(table, 0, 1)
    table_p = _tc_transpose(table_t)
    table_rows = table_p.reshape(2 * VOCAB, D)
    idx = (jnp.swapaxes(x, 0, 1) * 2).reshape(NJOBS, IW)
    out5 = _sc_embed(idx, table_rows)
    out = out5.transpose(2, 4, 0, 1, 3).reshape(BATCH, SEQ, D)
    return out


# speedup vs baseline: 3.4009x; 1.1183x over previous
import functools

import jax
import jax.numpy as jnp
from jax import lax
from jax.experimental import pallas as pl
from jax.experimental.pallas import tpu as pltpu
from jax.experimental.pallas import tpu_sc as plsc

VOCAB = 1000000
D = 64
DP = 128
BATCH = 4096
SEQ = 200
N = BATCH * SEQ
SCALE = 8.0

NC = 2
NS = 16
NW = NC * NS

IW = 128
NJOBS = N // IW
JOBS_W = NJOBS // NW
BH = BATCH // IW

_mesh = plsc.VectorSubcoreMesh(core_axis_name="c", subcore_axis_name="s")

TV = 8192


def _tt_body(x_ref, o_ref):
    x = x_ref[...]
    x2 = jnp.concatenate([x, x], axis=0) * SCALE
    o_ref[...] = lax.transpose(x2, (1, 0))


_tc_transpose = pl.pallas_call(
    _tt_body,
    grid=(pl.cdiv(VOCAB, TV),),
    in_specs=[pl.BlockSpec((D, TV), lambda i: (0, i))],
    out_specs=pl.BlockSpec((TV, DP), lambda i: (i, 0)),
    out_shape=jax.ShapeDtypeStruct((VOCAB, DP), jnp.float32),
)

_IOTA16 = None


@functools.partial(
    pl.kernel,
    mesh=_mesh,
    out_type=jax.ShapeDtypeStruct((SEQ, 8, BH, 8, IW), jnp.float32),
    compiler_params=pltpu.CompilerParams(
        use_tc_tiling_on_sc=False, needs_layout_passes=False),
    scratch_types=[
        pltpu.VMEM((JOBS_W, IW), jnp.int32),
        pltpu.VMEM((IW, D), jnp.float32),
        pltpu.VMEM((IW, D), jnp.float32),
        pltpu.VMEM((D, IW + 1), jnp.float32),
        pltpu.VMEM((D, IW + 1), jnp.float32),
        pltpu.SemaphoreType.DMA,
        pltpu.SemaphoreType.DMA,
        pltpu.SemaphoreType.DMA,
        pltpu.SemaphoreType.DMA,
        pltpu.SemaphoreType.DMA,
    ],
)
def _sc_embed(idx_hbm, table_hbm, out_hbm, idx_v, g_a, g_b, t_a, t_b,
              isem, gsem_a, gsem_b, ssem_a, ssem_b):
    wid = lax.axis_index("s") * NC + lax.axis_index("c")
    job0 = wid * JOBS_W
    pltpu.async_copy(idx_hbm.at[pl.ds(job0, JOBS_W)], idx_v, isem).wait()

    iota = lax.iota(jnp.int32, 16)

    def fire(t, buf, sem):
        pltpu.async_copy(table_hbm.at[idx_v.at[t]], buf, sem)

    def drain(t, buf, sem):
        pltpu.make_async_copy(table_hbm.at[idx_v.at[t]], buf, sem).wait()

    rows_c = [iota + c * 16 for c in range(D // 16)]

    def transpose_scale(gbuf, tbuf):
        @plsc.parallel_loop(0, IW, unroll=8)
        def _(r):
            col = jnp.full((16,), r, jnp.int32)
            for c in range(D // 16):
                v = gbuf.at[r][pl.ds(c * 16, 16)]
                plsc.store_scatter(tbuf, [rows_c[c], col], v)

    def fire_stores(t, tbuf, sem):
        j = job0 + t
        s = j // BH
        bh = j % BH
        for dh in range(8):
            pltpu.async_copy(
                tbuf.at[pl.ds(dh * 8, 8), pl.ds(0, IW)],
                out_hbm.at[s].at[dh].at[bh], sem)

    def wait_stores(tbuf, sem):
        for dh in range(8):
            pltpu.make_async_copy(
                tbuf.at[pl.ds(dh * 8, 8), pl.ds(0, IW)],
                out_hbm.at[0].at[dh].at[0], sem
            ).wait()

    fire(0, g_a, gsem_a)

    @pl.loop(0, JOBS_W // 2)
    def _(p):
        t0 = 2 * p
        fire(t0 + 1, g_b, gsem_b)
        drain(t0, g_a, gsem_a)

        @pl.when(p > 0)
        def _():
            wait_stores(t_a, ssem_a)

        transpose_scale(g_a, t_a)
        fire_stores(t0, t_a, ssem_a)

        @pl.when(t0 + 2 < JOBS_W)
        def _():
            fire(t0 + 2, g_a, gsem_a)

        drain(t0 + 1, g_b, gsem_b)

        @pl.when(p > 0)
        def _():
            wait_stores(t_b, ssem_b)

        transpose_scale(g_b, t_b)
        fire_stores(t0 + 1, t_b, ssem_b)

    wait_stores(t_a, ssem_a)
    wait_stores(t_b, ssem_b)


def kernel(x, table):
    table_t = jnp.swapaxes(table, 0, 1)
    table_p = _tc_transpose(table_t)
    table_rows = table_p.reshape(2 * VOCAB, D)
    idx = (jnp.swapaxes(x, 0, 1) * 2).reshape(NJOBS, IW)
    out5 = _sc_embed(idx, table_rows)
    out = out5.transpose(2, 4, 0, 1, 3).reshape(BATCH, SEQ, D)
    return out


# speedup vs baseline: 3.8818x; 1.1414x over previous
import functools

import jax
import jax.numpy as jnp
from jax import lax
from jax.experimental import pallas as pl
from jax.experimental.pallas import tpu as pltpu
from jax.experimental.pallas import tpu_sc as plsc

VOCAB = 1000000
D = 64
DP = 128
BATCH = 4096
SEQ = 200
N = BATCH * SEQ
SCALE = 8.0

NC = 2
NS = 16
NW = NC * NS

IW = 128
NJOBS = N // IW
JOBS_W = NJOBS // NW
BH = BATCH // IW

_mesh = plsc.VectorSubcoreMesh(core_axis_name="c", subcore_axis_name="s")

TV = 16384


def _tt_body(x_ref, o_ref):
    x = x_ref[...]
    x2 = jnp.concatenate([x, x], axis=0) * SCALE
    o_ref[...] = lax.transpose(x2, (1, 0))


_tc_transpose = pl.pallas_call(
    _tt_body,
    grid=(pl.cdiv(VOCAB, TV),),
    in_specs=[pl.BlockSpec((D, TV), lambda i: (0, i))],
    out_specs=pl.BlockSpec((TV, DP), lambda i: (i, 0)),
    out_shape=jax.ShapeDtypeStruct((VOCAB, DP), jnp.float32),
)

_IOTA16 = None


@functools.partial(
    pl.kernel,
    mesh=_mesh,
    out_type=jax.ShapeDtypeStruct((SEQ, 8, BH, 8, IW), jnp.float32),
    compiler_params=pltpu.CompilerParams(
        use_tc_tiling_on_sc=False, needs_layout_passes=False),
    scratch_types=[
        pltpu.VMEM((JOBS_W, IW), jnp.int32),
        pltpu.VMEM((4, IW, D), jnp.float32),
        pltpu.VMEM((4, D, IW + 1), jnp.float32),
        pltpu.SemaphoreType.DMA,
        pltpu.SemaphoreType.DMA,
        pltpu.SemaphoreType.DMA,
        pltpu.SemaphoreType.DMA,
        pltpu.SemaphoreType.DMA,
        pltpu.SemaphoreType.DMA,
        pltpu.SemaphoreType.DMA,
        pltpu.SemaphoreType.DMA,
        pltpu.SemaphoreType.DMA,
    ],
)
def _sc_embed(idx_hbm, table_hbm, out_hbm, idx_v, g4, t4, isem,
              gs0, gs1, gs2, gs3, ss0, ss1, ss2, ss3):
    wid = lax.axis_index("s") * NC + lax.axis_index("c")
    job0 = wid * JOBS_W
    pltpu.async_copy(idx_hbm.at[pl.ds(job0, JOBS_W)], idx_v, isem).wait()

    iota = lax.iota(jnp.int32, 16)
    gsem = [gs0, gs1, gs2, gs3]
    ssem = [ss0, ss1, ss2, ss3]

    def fire(t, q):
        pltpu.async_copy(table_hbm.at[idx_v.at[t]], g4.at[q], gsem[q])

    def drain(t, q):
        pltpu.make_async_copy(table_hbm.at[idx_v.at[t]], g4.at[q],
                              gsem[q]).wait()

    rows_c = [iota + c * 16 for c in range(D // 16)]

    def transpose_scale(q):
        gbuf = g4.at[q]
        tbuf = t4.at[q]

        @plsc.parallel_loop(0, IW, unroll=8)
        def _(r):
            col = jnp.full((16,), r, jnp.int32)
            for c in range(D // 16):
                v = gbuf.at[r][pl.ds(c * 16, 16)]
                plsc.store_scatter(tbuf, [rows_c[c], col], v)

    def fire_stores(t, q):
        j = job0 + t
        s = j // BH
        bh = j % BH
        for dh in range(8):
            pltpu.async_copy(
                t4.at[q].at[pl.ds(dh * 8, 8), pl.ds(0, IW)],
                out_hbm.at[s].at[dh].at[bh], ssem[q])

    def wait_stores(q):
        for dh in range(8):
            pltpu.make_async_copy(
                t4.at[q].at[pl.ds(dh * 8, 8), pl.ds(0, IW)],
                out_hbm.at[0].at[dh].at[0], ssem[q]
            ).wait()

    for q in range(4):
        fire(q, q)

    @pl.loop(0, JOBS_W // 4)
    def _(p):
        t0 = 4 * p
        for q in range(4):
            t = t0 + q
            drain(t, q)

            @pl.when(p > 0)
            def _():
                wait_stores(q)

            transpose_scale(q)
            fire_stores(t, q)

            @pl.when(t + 4 < JOBS_W)
            def _():
                fire(t + 4, q)

    for q in range(4):
        wait_stores(q)


def kernel(x, table):
    table_t = jnp.swapaxes(table, 0, 1)
    table_p = _tc_transpose(table_t)
    table_rows = table_p.reshape(2 * VOCAB, D)
    idx = (jnp.swapaxes(x, 0, 1) * 2).reshape(NJOBS, IW)
    out5 = _sc_embed(idx, table_rows)
    out = out5.transpose(2, 4, 0, 1, 3).reshape(BATCH, SEQ, D)
    return out


# speedup vs baseline: 4.7643x; 1.2273x over previous
import functools

import jax
import jax.numpy as jnp
from jax import lax
from jax.experimental import pallas as pl
from jax.experimental.pallas import tpu as pltpu
from jax.experimental.pallas import tpu_sc as plsc

VOCAB = 1000000
D = 64
DP = 128
BATCH = 4096
SEQ = 200
N = BATCH * SEQ
SCALE = 8.0

NC = 2
NS = 16
NW = NC * NS

IW = 128
NJOBS = N // IW
JOBS_W = NJOBS // NW
BH = BATCH // IW

_mesh = plsc.VectorSubcoreMesh(core_axis_name="c", subcore_axis_name="s")

TV = 8192


def _tt_body(xa_ref, xb_ref, o_ref):
    x2 = jnp.concatenate([xa_ref[...], xb_ref[...]], axis=0) * SCALE
    o_ref[...] = lax.transpose(x2, (1, 0))


_tc_transpose = pl.pallas_call(
    _tt_body,
    grid=(pl.cdiv(VOCAB, 2 * TV),),
    in_specs=[
        pl.BlockSpec((D, TV), lambda i: (0, 2 * i)),
        pl.BlockSpec((D, TV),
                     lambda i: (0, jnp.minimum(2 * i + 1, VOCAB // TV))),
    ],
    out_specs=pl.BlockSpec((TV, DP), lambda i: (i, 0)),
    out_shape=jax.ShapeDtypeStruct((VOCAB // 2, DP), jnp.float32),
)

_IOTA16 = None


@functools.partial(
    pl.kernel,
    mesh=_mesh,
    out_type=jax.ShapeDtypeStruct((SEQ, 8, BH, 8, IW), jnp.float32),
    compiler_params=pltpu.CompilerParams(
        use_tc_tiling_on_sc=False, needs_layout_passes=False),
    scratch_types=[
        pltpu.VMEM((JOBS_W, IW), jnp.int32),
        pltpu.VMEM((4, IW, D), jnp.float32),
        pltpu.VMEM((4, D, IW + 1), jnp.float32),
        pltpu.SemaphoreType.DMA,
        pltpu.SemaphoreType.DMA,
        pltpu.SemaphoreType.DMA,
        pltpu.SemaphoreType.DMA,
        pltpu.SemaphoreType.DMA,
        pltpu.SemaphoreType.DMA,
        pltpu.SemaphoreType.DMA,
        pltpu.SemaphoreType.DMA,
        pltpu.SemaphoreType.DMA,
    ],
)
def _sc_embed(idx_hbm, table_hbm, out_hbm, idx_v, g4, t4, isem,
              gs0, gs1, gs2, gs3, ss0, ss1, ss2, ss3):
    wid = lax.axis_index("s") * NC + lax.axis_index("c")
    job0 = wid * JOBS_W
    pltpu.async_copy(idx_hbm.at[pl.ds(job0, JOBS_W)], idx_v, isem).wait()

    iota = lax.iota(jnp.int32, 16)
    gsem = [gs0, gs1, gs2, gs3]
    ssem = [ss0, ss1, ss2, ss3]

    def fire(t, q):
        pltpu.async_copy(table_hbm.at[idx_v.at[t]], g4.at[q], gsem[q])

    def drain(t, q):
        pltpu.make_async_copy(table_hbm.at[idx_v.at[t]], g4.at[q],
                              gsem[q]).wait()

    rows_c = [iota + c * 16 for c in range(D // 16)]

    def transpose_scale(q):
        gbuf = g4.at[q]
        tbuf = t4.at[q]

        @plsc.parallel_loop(0, IW, unroll=8)
        def _(r):
            col = jnp.full((16,), r, jnp.int32)
            for c in range(D // 16):
                v = gbuf.at[r][pl.ds(c * 16, 16)]
                plsc.store_scatter(tbuf, [rows_c[c], col], v)

    def fire_stores(t, q):
        j = job0 + t
        s = j // BH
        bh = j % BH
        for dh in range(8):
            pltpu.async_copy(
                t4.at[q].at[pl.ds(dh * 8, 8), pl.ds(0, IW)],
                out_hbm.at[s].at[dh].at[bh], ssem[q])

    def wait_stores(q):
        for dh in range(8):
            pltpu.make_async_copy(
                t4.at[q].at[pl.ds(dh * 8, 8), pl.ds(0, IW)],
                out_hbm.at[0].at[dh].at[0], ssem[q]
            ).wait()

    for q in range(4):
        fire(q, q)

    @pl.loop(0, JOBS_W // 4)
    def _(p):
        t0 = 4 * p
        for q in range(4):
            t = t0 + q
            drain(t, q)

            @pl.when(p > 0)
            def _():
                wait_stores(q)

            transpose_scale(q)
            fire_stores(t, q)

            @pl.when(t + 4 < JOBS_W)
            def _():
                fire(t + 4, q)

    for q in range(4):
        wait_stores(q)


def kernel(x, table):
    table_t = jnp.swapaxes(table, 0, 1)
    table_p = _tc_transpose(table_t, table_t)
    table_rows = table_p.reshape(VOCAB, D)
    xt = jnp.swapaxes(x, 0, 1)
    g, r = xt // (2 * TV), xt % (2 * TV)
    flat = g * (2 * TV) + jnp.where(r < TV, 2 * r, 2 * (r - TV) + 1)
    idx = flat.reshape(NJOBS, IW)
    out5 = _sc_embed(idx, table_rows)
    out = out5.transpose(2, 4, 0, 1, 3).reshape(BATCH, SEQ, D)
    return out
